# baseline (speedup 1.0000x reference)
import functools

import jax
import jax.numpy as jnp
from jax.experimental import pallas as pl
from jax.experimental.pallas import tpu as pltpu

K = 32


def _enc_topk_kernel(x_ref, w_ref, b_ref, out_ref, z_s, *, nd, dt):
    j = pl.program_id(1)
    x = x_ref[...]
    w = w_ref[...]
    z = jax.lax.dot_general(x, w, (((1,), (1,)), ((), ())),
                            preferred_element_type=jnp.float32)
    z = z + b_ref[...]
    z_s[j] = z

    @pl.when(j == nd - 1)
    def _():
        zv = z_s[...]
        tb = zv.shape[1]

        qpt = -(-K // nd)
        cw = dt // qpt
        cms = []
        for jj in range(nd):
            zj = z_s[jj]
            for q in range(qpt):
                cms.append(jnp.max(zj[:, q * cw:(q + 1) * cw], axis=1,
                                   keepdims=True))
        lo0 = cms[0]
        hi0 = cms[0]
        for c in cms[1:]:
            lo0 = jnp.minimum(lo0, c)
            hi0 = jnp.maximum(hi0, c)

        kf = jnp.float32(K)

        def cond(c):
            i, lo, hi, cl = c
            return jnp.logical_and(i < 40, jnp.any(cl != kf))

        def body(c):
            i, lo, hi, cl = c
            mid = 0.5 * (lo + hi)
            m = (zv >= mid[None, :, :]).astype(jnp.float32)
            c1 = jnp.sum(m, axis=2)
            cnt = jnp.sum(c1, axis=0)[:, None]
            ge = cnt >= kf
            return (i + 1,
                    jnp.where(ge, mid, lo),
                    jnp.where(ge, hi, mid),
                    jnp.where(ge, cnt, cl))

        _, thr, _, _ = jax.lax.while_loop(
            cond, body,
            (jnp.int32(0), lo0, hi0, jnp.full((tb, 1), kf + 1.0, jnp.float32)))

        for jj in range(nd):
            zj = z_s[jj]
            out_ref[:, jj * dt:(jj + 1) * dt] = jnp.where(zj >= thr, zj, 0.0)


def _dec_kernel(zs_ref, w_ref, b_ref, out_ref, acc, *, nd):
    j = pl.program_id(1)

    @pl.when(j == 0)
    def _():
        acc[...] = jnp.zeros_like(acc)

    acc[...] += jax.lax.dot_general(zs_ref[...], w_ref[...],
                                    (((1,), (1,)), ((), ())),
                                    preferred_element_type=jnp.float32)

    @pl.when(j == nd - 1)
    def _():
        out_ref[...] = acc[...] + b_ref[...]


def kernel(x, W_enc, b_enc, W_dec, b_dec):
    n_tok, d_in = x.shape
    d_dict = W_enc.shape[0]
    tb = min(256, n_tok)
    dt = 1024
    nt = n_tok // tb
    nd = d_dict // dt
    b_enc2 = b_enc.reshape(1, d_dict)
    b_dec2 = b_dec.reshape(1, d_in)

    z_sparse = pl.pallas_call(
        functools.partial(_enc_topk_kernel, nd=nd, dt=dt),
        grid=(nt, nd),
        in_specs=[
            pl.BlockSpec((tb, d_in), lambda i, j: (i, 0)),
            pl.BlockSpec((dt, d_in), lambda i, j: (j, 0)),
            pl.BlockSpec((1, dt), lambda i, j: (0, j)),
        ],
        out_specs=pl.BlockSpec((tb, d_dict), lambda i, j: (i, 0)),
        out_shape=jax.ShapeDtypeStruct((n_tok, d_dict), jnp.float32),
        scratch_shapes=[pltpu.VMEM((nd, tb, dt), jnp.float32)],
    )(x, W_enc, b_enc2)

    x_hat = pl.pallas_call(
        functools.partial(_dec_kernel, nd=nd),
        grid=(nt, nd),
        in_specs=[
            pl.BlockSpec((tb, dt), lambda i, j: (i, j)),
            pl.BlockSpec((d_in, dt), lambda i, j: (0, j)),
            pl.BlockSpec((1, d_in), lambda i, j: (0, 0)),
        ],
        out_specs=pl.BlockSpec((tb, d_in), lambda i, j: (i, 0)),
        out_shape=jax.ShapeDtypeStruct((n_tok, d_in), jnp.float32),
        scratch_shapes=[pltpu.VMEM((tb, d_in), jnp.float32)],
    )(z_sparse, W_dec, b_dec2)

    return (x_hat, z_sparse)
